# baseline (speedup 1.0000x reference)
import functools

import jax
import jax.numpy as jnp
from jax import lax
from jax.experimental import pallas as pl
from jax.experimental.pallas import tpu as pltpu
from jax.experimental.pallas import tpu_sc as plsc

_N = 10000
_NPAD = 10240
_C = 128
_NC = 2
_NS = 16
_NW = _NC * _NS
_RPT = _NPAD // _NS
_RB = 512
_G = _NPAD // _RB
_SPLIT = (64, 16)
_ZR = 64


def _sc_edge_accum(width, chunk_split, gather, ring):
    mesh = plsc.VectorSubcoreMesh(core_axis_name="c", subcore_axis_name="s")
    ch0, ch1 = chunk_split
    assert ch0 % ring == 0 and ch1 % ring == 0

    @functools.partial(
        pl.kernel,
        out_type=jax.ShapeDtypeStruct((2 * _NPAD, width), jnp.float32),
        mesh=mesh,
        scratch_types=[
            pltpu.VMEM((ring, 2, _C), jnp.int32),
            pltpu.VMEM((ring if gather else 1, _C, width), jnp.float32),
            pltpu.VMEM((_ZR, width), jnp.float32),
            pltpu.VMEM_SHARED((_NPAD, width), jnp.float32),
            pltpu.SemaphoreType.DMA,
            pltpu.SemaphoreType.DMA,
        ],
        compiler_params=pltpu.CompilerParams(use_tc_tiling_on_sc=False),
    )
    def k(ep_hbm, u_hbm, z_hbm, out_hbm, idx_v, rows_v, zb_v, acc_sh, gsem, ssem):
        cid = lax.axis_index("c")
        sid = lax.axis_index("s")
        r0 = sid * _RPT
        pltpu.sync_copy(z_hbm, zb_v)

        @pl.loop(0, _RPT // _ZR)
        def _(j):
            pltpu.sync_copy(zb_v, acc_sh.at[pl.ds(r0 + j * _ZR, _ZR)])
        if not gather:
            pltpu.sync_copy(u_hbm, rows_v.at[0])
        plsc.subcore_barrier()

        chc = jnp.where(cid == 0, ch0, ch1)
        cbase = cid * (_NS * ch0) + sid * chc

        @pl.loop(0, chc // ring)
        def _(s):
            base = cbase + s * ring
            pltpu.sync_copy(ep_hbm.at[pl.ds(base, ring)], idx_v)
            if gather:
                gds = [
                    pltpu.async_copy(u_hbm.at[idx_v.at[r, 0]],
                                     rows_v.at[r], gsem)
                    for r in range(ring)
                ]
            sds = []
            for r in range(ring):
                if gather:
                    gds[r].wait()
                    src = rows_v.at[r]
                else:
                    src = rows_v.at[0]
                sds.append(pltpu.async_copy(src, acc_sh.at[idx_v.at[r, 1]],
                                            ssem, add=True))
            for d in sds:
                d.wait()

        plsc.subcore_barrier()
        pltpu.sync_copy(acc_sh.at[pl.ds(r0, _RPT)],
                        out_hbm.at[pl.ds(cid * _NPAD + r0, _RPT)])

    return k


def _sc_agg_staged(wh, ch_all, ring):
    mesh = plsc.VectorSubcoreMesh(core_axis_name="c", subcore_axis_name="s")
    assert ch_all % ring == 0

    @functools.partial(
        pl.kernel,
        out_type=jax.ShapeDtypeStruct((2 * _NPAD, wh), jnp.float32),
        mesh=mesh,
        scratch_types=[
            pltpu.VMEM((ring, 2, _C), jnp.int32),
            pltpu.VMEM((ring, _C, wh), jnp.float32),
            pltpu.VMEM((_ZR, wh), jnp.float32),
            pltpu.VMEM_SHARED((_NPAD, wh), jnp.float32),
            pltpu.VMEM_SHARED((_NPAD, wh), jnp.float32),
            pltpu.SemaphoreType.DMA,
            pltpu.SemaphoreType.DMA,
        ],
        compiler_params=pltpu.CompilerParams(use_tc_tiling_on_sc=False),
    )
    def k(ep_hbm, ua_hbm, ub_hbm, z_hbm, out_hbm,
          idx_v, rows_v, zb_v, stage_sh, acc_sh, gsem, ssem):
        cid = lax.axis_index("c")
        sid = lax.axis_index("s")
        r0 = sid * _RPT

        @pl.when(cid == 0)
        def _():
            pltpu.sync_copy(ua_hbm.at[pl.ds(r0, _RPT)],
                            stage_sh.at[pl.ds(r0, _RPT)])

        @pl.when(cid == 1)
        def _():
            pltpu.sync_copy(ub_hbm.at[pl.ds(r0, _RPT)],
                            stage_sh.at[pl.ds(r0, _RPT)])

        pltpu.sync_copy(z_hbm, zb_v)

        @pl.loop(0, _RPT // _ZR)
        def _(j):
            pltpu.sync_copy(zb_v, acc_sh.at[pl.ds(r0 + j * _ZR, _ZR)])

        plsc.subcore_barrier()

        cbase = sid * ch_all

        @pl.loop(0, ch_all // ring)
        def _(s):
            base = cbase + s * ring
            pltpu.sync_copy(ep_hbm.at[pl.ds(base, ring)], idx_v)
            gds = [
                pltpu.async_copy(stage_sh.at[idx_v.at[r, 0]],
                                 rows_v.at[r], gsem)
                for r in range(ring)
            ]
            sds = []
            for r in range(ring):
                gds[r].wait()
                sds.append(pltpu.async_copy(rows_v.at[r],
                                            acc_sh.at[idx_v.at[r, 1]],
                                            ssem, add=True))
            for d in sds:
                d.wait()

        plsc.subcore_barrier()
        pltpu.sync_copy(acc_sh.at[pl.ds(r0, _RPT)],
                        out_hbm.at[pl.ds(cid * _NPAD + r0, _RPT)])

    return k


def _deg_spec(i):
    return (i, 0)


def _deg_spec2(i):
    return (_G + i, 0)


def _tc_h0(x_pad, Wb):
    def body(x_ref, w_ref, o_ref):
        o_ref[...] = jnp.dot(x_ref[...], w_ref[...],
                             preferred_element_type=jnp.float32)

    return pl.pallas_call(
        body,
        grid=(_G,),
        in_specs=[
            pl.BlockSpec((_RB, 128), lambda i: (i, 0)),
            pl.BlockSpec((128, 64), lambda i: (0, 0)),
        ],
        out_specs=pl.BlockSpec((_RB, 64), lambda i: (i, 0)),
        out_shape=jax.ShapeDtypeStruct((_NPAD, 64), jnp.float32),
    )(x_pad, Wb)


def _tc_scale(h0, degp):
    def body(h_ref, d0, d1, oa_ref, ob_ref):
        dinv = lax.rsqrt(1.0 + d0[:, 0:1] + d1[:, 0:1])
        u = h_ref[...] * dinv
        oa_ref[...] = u[:, :32]
        ob_ref[...] = u[:, 32:]

    return pl.pallas_call(
        body,
        grid=(_G,),
        in_specs=[
            pl.BlockSpec((_RB, 64), lambda i: (i, 0)),
            pl.BlockSpec((_RB, 16), _deg_spec),
            pl.BlockSpec((_RB, 16), _deg_spec2),
        ],
        out_specs=[
            pl.BlockSpec((_RB, 32), lambda i: (i, 0)),
            pl.BlockSpec((_RB, 32), lambda i: (i, 0)),
        ],
        out_shape=[
            jax.ShapeDtypeStruct((_NPAD, 32), jnp.float32),
            jax.ShapeDtypeStruct((_NPAD, 32), jnp.float32),
        ],
    )(h0, degp, degp)


def _tc_mid(agg0, u0a, u0b, degp, bb, gbe, beb, Wm, Ws):
    def body(aa, ab, ua, ub, d0, d1, bb_r, g_r, be_r, wm_r, ws_r,
             oa_ref, ob_ref):
        dinv = lax.rsqrt(1.0 + d0[:, 0:1] + d1[:, 0:1])
        agg = jnp.concatenate([aa[...] + ua[...], ab[...] + ub[...]], axis=1)
        s = dinv * agg + bb_r[...]
        hid = jax.nn.sigmoid(s) * g_r[...] + be_r[...]
        oa_ref[...] = jnp.dot(hid, wm_r[...],
                              preferred_element_type=jnp.float32) * dinv
        ob_ref[...] = jnp.dot(hid, ws_r[...],
                              preferred_element_type=jnp.float32) * dinv

    return pl.pallas_call(
        body,
        grid=(_G,),
        in_specs=[
            pl.BlockSpec((_RB, 32), _deg_spec),
            pl.BlockSpec((_RB, 32), _deg_spec2),
            pl.BlockSpec((_RB, 32), lambda i: (i, 0)),
            pl.BlockSpec((_RB, 32), lambda i: (i, 0)),
            pl.BlockSpec((_RB, 16), _deg_spec),
            pl.BlockSpec((_RB, 16), _deg_spec2),
            pl.BlockSpec((1, 64), lambda i: (0, 0)),
            pl.BlockSpec((1, 64), lambda i: (0, 0)),
            pl.BlockSpec((1, 64), lambda i: (0, 0)),
            pl.BlockSpec((64, 64), lambda i: (0, 0)),
            pl.BlockSpec((64, 64), lambda i: (0, 0)),
        ],
        out_specs=[
            pl.BlockSpec((_RB, 64), lambda i: (i, 0)),
            pl.BlockSpec((_RB, 64), lambda i: (i, 0)),
        ],
        out_shape=[
            jax.ShapeDtypeStruct((_NPAD, 64), jnp.float32),
            jax.ShapeDtypeStruct((_NPAD, 64), jnp.float32),
        ],
    )(agg0, agg0, u0a, u0b, degp, degp, bb, gbe, beb, Wm, Ws)


def _tc_head(agg1, u1a, u1b, degp, noise, bm, gme, bem, bs, gse, bes,
             D1, db1, D2, db2):
    def body(am, al, ua, ub, d0, d1, nz, bm_r, gm_r, bem_r, bs_r, gs_r, bes_r,
             d1_r, db1_r, d2_r, db2_r, z_ref, rec_ref):
        dinv = lax.rsqrt(1.0 + d0[:, 0:1] + d1[:, 0:1])
        tm = dinv * (am[...] + ua[...])
        tl = dinv * (al[...] + ub[...])
        mean = jax.nn.sigmoid(tm + bm_r[...]) * gm_r[...] + bem_r[...]
        ls = jax.nn.sigmoid(tl + bs_r[...]) * gs_r[...] + bes_r[...]
        z = nz[...] * jnp.exp(ls) + mean
        z_ref[...] = z
        a = jnp.dot(z, d1_r[...], preferred_element_type=jnp.float32) + db1_r[...]
        a = jnp.where(a > 0, a, 0.01 * a)
        rec_ref[...] = jnp.dot(a, d2_r[...],
                               preferred_element_type=jnp.float32) + db2_r[...]

    vec64 = pl.BlockSpec((1, 64), lambda i: (0, 0))
    return pl.pallas_call(
        body,
        grid=(_G,),
        in_specs=[
            pl.BlockSpec((_RB, 64), _deg_spec),
            pl.BlockSpec((_RB, 64), _deg_spec2),
            pl.BlockSpec((_RB, 64), lambda i: (i, 0)),
            pl.BlockSpec((_RB, 64), lambda i: (i, 0)),
            pl.BlockSpec((_RB, 16), _deg_spec),
            pl.BlockSpec((_RB, 16), _deg_spec2),
            pl.BlockSpec((_RB, 64), lambda i: (i, 0)),
            vec64, vec64, vec64, vec64, vec64, vec64,
            pl.BlockSpec((64, 64), lambda i: (0, 0)),
            vec64,
            pl.BlockSpec((64, 128), lambda i: (0, 0)),
            pl.BlockSpec((1, 128), lambda i: (0, 0)),
        ],
        out_specs=[
            pl.BlockSpec((_RB, 64), lambda i: (i, 0)),
            pl.BlockSpec((_RB, 128), lambda i: (i, 0)),
        ],
        out_shape=[
            jax.ShapeDtypeStruct((_NPAD, 64), jnp.float32),
            jax.ShapeDtypeStruct((_NPAD, 128), jnp.float32),
        ],
    )(agg1, agg1, u1a, u1b, degp, degp, noise, bm, gme, bem, bs, gse, bes,
      D1, db1, D2, db2)


def _tc_decode(z_pad, zT):
    def body(zi, za, o_ref):
        o_ref[...] = jax.nn.sigmoid(
            jnp.dot(zi[...], za[...], preferred_element_type=jnp.float32))

    return pl.pallas_call(
        body,
        grid=(_N // 400,),
        in_specs=[
            pl.BlockSpec((400, 64), lambda i: (i, 0)),
            pl.BlockSpec((64, _N), lambda i: (0, 0)),
        ],
        out_specs=pl.BlockSpec((400, _N), lambda i: (i, 0)),
        out_shape=jax.ShapeDtypeStruct((_N, _N), jnp.float32),
    )(z_pad, zT)


def kernel(x, edge_index, batch, Wb, bb, gb, beb, Wm, bm, gm, bem,
           Ws, bs, gs, bes, D1, db1, D2, db2):
    n, f_in = x.shape
    e = edge_index.shape[1]
    ch = -(-e // (_C * _NW))
    e_pad = ch * _C * _NW

    x_pad = jnp.pad(x, ((0, _NPAD - n), (0, 0)))
    pad = jnp.full((e_pad - e,), n, jnp.int32)
    srcp = jnp.concatenate([edge_index[0].astype(jnp.int32), pad])
    dstp = jnp.concatenate([edge_index[1].astype(jnp.int32), pad])
    ep = jnp.stack([srcp.reshape(-1, _C), dstp.reshape(-1, _C)], axis=1)

    inv_c = 1.0 / jnp.sqrt(1.0 + 1e-4)
    gbe = (gb * inv_c).reshape(1, 64)
    gme = (gm * inv_c).reshape(1, 64)
    gse = (gs * inv_c).reshape(1, 64)
    bb2, beb2 = bb.reshape(1, 64), beb.reshape(1, 64)
    bm2, bem2 = bm.reshape(1, 64), bem.reshape(1, 64)
    bs2, bes2 = bs.reshape(1, 64), bes.reshape(1, 64)
    db1_2, db2_2 = db1.reshape(1, 64), db2.reshape(1, 128)

    noise = jax.random.normal(jax.random.key(42), (n, 64), jnp.float32)
    noise = jnp.pad(noise, ((0, _NPAD - n), (0, 0)))

    ones16 = jnp.ones((_C, 16), jnp.float32)
    z16 = jnp.zeros((_ZR, 16), jnp.float32)
    z32 = jnp.zeros((_ZR, 32), jnp.float32)
    z64 = jnp.zeros((_ZR, 64), jnp.float32)

    degp = _sc_edge_accum(16, (ch, ch), gather=False, ring=8)(ep, ones16, z16)

    h0 = _tc_h0(x_pad, Wb)
    u0a, u0b = _tc_scale(h0, degp)

    agg0 = _sc_agg_staged(32, 2 * ch, ring=8)(ep, u0a, u0b, z32)

    u1a, u1b = _tc_mid(agg0, u0a, u0b, degp, bb2, gbe, beb2, Wm, Ws)

    agg1 = _sc_agg_staged(64, 2 * ch, ring=5)(ep, u1a, u1b, z64)

    z_pad, rec_pad = _tc_head(agg1, u1a, u1b, degp, noise, bm2, gme, bem2,
                              bs2, gse, bes2, D1, db1_2, D2, db2_2)

    rec = rec_pad[:n]
    zT = z_pad[:n].T
    a_pred = _tc_decode(z_pad, zT)
    return rec, a_pred


# speedup vs baseline: 1.0039x; 1.0039x over previous
import functools

import jax
import jax.numpy as jnp
from jax import lax
from jax.experimental import pallas as pl
from jax.experimental.pallas import tpu as pltpu
from jax.experimental.pallas import tpu_sc as plsc

_N = 10000
_NPAD = 10240
_C = 128
_NC = 2
_NS = 16
_NW = _NC * _NS
_RPT = _NPAD // _NS
_RB = 512
_G = _NPAD // _RB
_SPLIT = (64, 16)
_ZR = 64


def _sc_edge_accum(width, chunk_split, gather, ring):
    mesh = plsc.VectorSubcoreMesh(core_axis_name="c", subcore_axis_name="s")
    ch0, ch1 = chunk_split
    assert ch0 % ring == 0 and ch1 % ring == 0

    @functools.partial(
        pl.kernel,
        out_type=jax.ShapeDtypeStruct((2 * _NPAD, width), jnp.float32),
        mesh=mesh,
        scratch_types=[
            pltpu.VMEM((ring, 2, _C), jnp.int32),
            pltpu.VMEM((ring if gather else 1, _C, width), jnp.float32),
            pltpu.VMEM((_ZR, width), jnp.float32),
            pltpu.VMEM_SHARED((_NPAD, width), jnp.float32),
            pltpu.SemaphoreType.DMA,
            pltpu.SemaphoreType.DMA,
        ],
        compiler_params=pltpu.CompilerParams(use_tc_tiling_on_sc=False),
    )
    def k(ep_hbm, u_hbm, z_hbm, out_hbm, idx_v, rows_v, zb_v, acc_sh, gsem, ssem):
        cid = lax.axis_index("c")
        sid = lax.axis_index("s")
        r0 = sid * _RPT
        pltpu.sync_copy(z_hbm, zb_v)

        @pl.loop(0, _RPT // _ZR)
        def _(j):
            pltpu.sync_copy(zb_v, acc_sh.at[pl.ds(r0 + j * _ZR, _ZR)])
        if not gather:
            pltpu.sync_copy(u_hbm, rows_v.at[0])
        plsc.subcore_barrier()

        chc = jnp.where(cid == 0, ch0, ch1)
        cbase = cid * (_NS * ch0) + sid * chc

        @pl.loop(0, chc // ring)
        def _(s):
            base = cbase + s * ring
            pltpu.sync_copy(ep_hbm.at[pl.ds(base, ring)], idx_v)
            if gather:
                gds = [
                    pltpu.async_copy(u_hbm.at[idx_v.at[r, 0]],
                                     rows_v.at[r], gsem)
                    for r in range(ring)
                ]
            sds = []
            for r in range(ring):
                if gather:
                    gds[r].wait()
                    src = rows_v.at[r]
                else:
                    src = rows_v.at[0]
                sds.append(pltpu.async_copy(src, acc_sh.at[idx_v.at[r, 1]],
                                            ssem, add=True))
            for d in sds:
                d.wait()

        plsc.subcore_barrier()
        pltpu.sync_copy(acc_sh.at[pl.ds(r0, _RPT)],
                        out_hbm.at[pl.ds(cid * _NPAD + r0, _RPT)])

    return k


def _sc_agg_staged(wh, ch_all, ring):
    mesh = plsc.VectorSubcoreMesh(core_axis_name="c", subcore_axis_name="s")
    assert ch_all % ring == 0

    @functools.partial(
        pl.kernel,
        out_type=jax.ShapeDtypeStruct((2 * _NPAD, wh), jnp.float32),
        mesh=mesh,
        scratch_types=[
            pltpu.VMEM((ring, 2, _C), jnp.int32),
            pltpu.VMEM((ring, _C, wh), jnp.float32),
            pltpu.VMEM((_ZR, wh), jnp.float32),
            pltpu.VMEM_SHARED((_NPAD, wh), jnp.float32),
            pltpu.VMEM_SHARED((_NPAD, wh), jnp.float32),
            pltpu.SemaphoreType.DMA,
            pltpu.SemaphoreType.DMA,
        ],
        compiler_params=pltpu.CompilerParams(use_tc_tiling_on_sc=False),
    )
    def k(ep_hbm, ua_hbm, ub_hbm, z_hbm, out_hbm,
          idx_v, rows_v, zb_v, stage_sh, acc_sh, gsem, ssem):
        cid = lax.axis_index("c")
        sid = lax.axis_index("s")
        r0 = sid * _RPT

        @pl.when(cid == 0)
        def _():
            pltpu.sync_copy(ua_hbm.at[pl.ds(r0, _RPT)],
                            stage_sh.at[pl.ds(r0, _RPT)])

        @pl.when(cid == 1)
        def _():
            pltpu.sync_copy(ub_hbm.at[pl.ds(r0, _RPT)],
                            stage_sh.at[pl.ds(r0, _RPT)])

        pltpu.sync_copy(z_hbm, zb_v)

        @pl.loop(0, _RPT // _ZR)
        def _(j):
            pltpu.sync_copy(zb_v, acc_sh.at[pl.ds(r0 + j * _ZR, _ZR)])

        plsc.subcore_barrier()

        cbase = sid * ch_all

        @pl.loop(0, ch_all // ring)
        def _(s):
            base = cbase + s * ring
            pltpu.sync_copy(ep_hbm.at[pl.ds(base, ring)], idx_v)
            gds = [
                pltpu.async_copy(stage_sh.at[idx_v.at[r, 0]],
                                 rows_v.at[r], gsem)
                for r in range(ring)
            ]
            sds = []
            for r in range(ring):
                gds[r].wait()
                sds.append(pltpu.async_copy(rows_v.at[r],
                                            acc_sh.at[idx_v.at[r, 1]],
                                            ssem, add=True))
            for d in sds:
                d.wait()

        plsc.subcore_barrier()
        pltpu.sync_copy(acc_sh.at[pl.ds(r0, _RPT)],
                        out_hbm.at[pl.ds(cid * _NPAD + r0, _RPT)])

    return k


def _deg_spec(i):
    return (i, 0)


def _deg_spec2(i):
    return (_G + i, 0)


def _tc_h0(x_pad, Wb):
    def body(x_ref, w_ref, o_ref):
        o_ref[...] = jnp.dot(x_ref[...], w_ref[...],
                             preferred_element_type=jnp.float32)

    return pl.pallas_call(
        body,
        grid=(_G,),
        in_specs=[
            pl.BlockSpec((_RB, 128), lambda i: (i, 0)),
            pl.BlockSpec((128, 64), lambda i: (0, 0)),
        ],
        out_specs=pl.BlockSpec((_RB, 64), lambda i: (i, 0)),
        out_shape=jax.ShapeDtypeStruct((_NPAD, 64), jnp.float32),
    )(x_pad, Wb)


def _tc_scale(h0, degp):
    def body(h_ref, d0, d1, oa_ref, ob_ref):
        dinv = lax.rsqrt(1.0 + d0[:, 0:1] + d1[:, 0:1])
        u = h_ref[...] * dinv
        oa_ref[...] = u[:, :32]
        ob_ref[...] = u[:, 32:]

    return pl.pallas_call(
        body,
        grid=(_G,),
        in_specs=[
            pl.BlockSpec((_RB, 64), lambda i: (i, 0)),
            pl.BlockSpec((_RB, 16), _deg_spec),
            pl.BlockSpec((_RB, 16), _deg_spec2),
        ],
        out_specs=[
            pl.BlockSpec((_RB, 32), lambda i: (i, 0)),
            pl.BlockSpec((_RB, 32), lambda i: (i, 0)),
        ],
        out_shape=[
            jax.ShapeDtypeStruct((_NPAD, 32), jnp.float32),
            jax.ShapeDtypeStruct((_NPAD, 32), jnp.float32),
        ],
    )(h0, degp, degp)


def _tc_mid(agg0, u0a, u0b, degp, bb, gbe, beb, Wm, Ws):
    def body(aa, ab, ua, ub, d0, d1, bb_r, g_r, be_r, wm_r, ws_r,
             oa_ref, ob_ref):
        dinv = lax.rsqrt(1.0 + d0[:, 0:1] + d1[:, 0:1])
        agg = jnp.concatenate([aa[...] + ua[...], ab[...] + ub[...]], axis=1)
        s = dinv * agg + bb_r[...]
        hid = jax.nn.sigmoid(s) * g_r[...] + be_r[...]
        oa_ref[...] = jnp.dot(hid, wm_r[...],
                              preferred_element_type=jnp.float32) * dinv
        ob_ref[...] = jnp.dot(hid, ws_r[...],
                              preferred_element_type=jnp.float32) * dinv

    return pl.pallas_call(
        body,
        grid=(_G,),
        in_specs=[
            pl.BlockSpec((_RB, 32), _deg_spec),
            pl.BlockSpec((_RB, 32), _deg_spec2),
            pl.BlockSpec((_RB, 32), lambda i: (i, 0)),
            pl.BlockSpec((_RB, 32), lambda i: (i, 0)),
            pl.BlockSpec((_RB, 16), _deg_spec),
            pl.BlockSpec((_RB, 16), _deg_spec2),
            pl.BlockSpec((1, 64), lambda i: (0, 0)),
            pl.BlockSpec((1, 64), lambda i: (0, 0)),
            pl.BlockSpec((1, 64), lambda i: (0, 0)),
            pl.BlockSpec((64, 64), lambda i: (0, 0)),
            pl.BlockSpec((64, 64), lambda i: (0, 0)),
        ],
        out_specs=[
            pl.BlockSpec((_RB, 64), lambda i: (i, 0)),
            pl.BlockSpec((_RB, 64), lambda i: (i, 0)),
        ],
        out_shape=[
            jax.ShapeDtypeStruct((_NPAD, 64), jnp.float32),
            jax.ShapeDtypeStruct((_NPAD, 64), jnp.float32),
        ],
    )(agg0, agg0, u0a, u0b, degp, degp, bb, gbe, beb, Wm, Ws)


def _tc_head(agg1, u1a, u1b, degp, noise, bm, gme, bem, bs, gse, bes,
             D1, db1, D2, db2):
    def body(am, al, ua, ub, d0, d1, nz, bm_r, gm_r, bem_r, bs_r, gs_r, bes_r,
             d1_r, db1_r, d2_r, db2_r, z_ref, rec_ref):
        dinv = lax.rsqrt(1.0 + d0[:, 0:1] + d1[:, 0:1])
        tm = dinv * (am[...] + ua[...])
        tl = dinv * (al[...] + ub[...])
        mean = jax.nn.sigmoid(tm + bm_r[...]) * gm_r[...] + bem_r[...]
        ls = jax.nn.sigmoid(tl + bs_r[...]) * gs_r[...] + bes_r[...]
        z = nz[...] * jnp.exp(ls) + mean
        z_ref[...] = z
        a = jnp.dot(z, d1_r[...], preferred_element_type=jnp.float32) + db1_r[...]
        a = jnp.where(a > 0, a, 0.01 * a)
        rec_ref[...] = jnp.dot(a, d2_r[...],
                               preferred_element_type=jnp.float32) + db2_r[...]

    vec64 = pl.BlockSpec((1, 64), lambda i: (0, 0))
    return pl.pallas_call(
        body,
        grid=(_G,),
        in_specs=[
            pl.BlockSpec((_RB, 64), _deg_spec),
            pl.BlockSpec((_RB, 64), _deg_spec2),
            pl.BlockSpec((_RB, 64), lambda i: (i, 0)),
            pl.BlockSpec((_RB, 64), lambda i: (i, 0)),
            pl.BlockSpec((_RB, 16), _deg_spec),
            pl.BlockSpec((_RB, 16), _deg_spec2),
            pl.BlockSpec((_RB, 64), lambda i: (i, 0)),
            vec64, vec64, vec64, vec64, vec64, vec64,
            pl.BlockSpec((64, 64), lambda i: (0, 0)),
            vec64,
            pl.BlockSpec((64, 128), lambda i: (0, 0)),
            pl.BlockSpec((1, 128), lambda i: (0, 0)),
        ],
        out_specs=[
            pl.BlockSpec((_RB, 64), lambda i: (i, 0)),
            pl.BlockSpec((_RB, 128), lambda i: (i, 0)),
        ],
        out_shape=[
            jax.ShapeDtypeStruct((_NPAD, 64), jnp.float32),
            jax.ShapeDtypeStruct((_NPAD, 128), jnp.float32),
        ],
    )(agg1, agg1, u1a, u1b, degp, degp, noise, bm, gme, bem, bs, gse, bes,
      D1, db1, D2, db2)


def _tc_decode(z_pad, zT):
    def body(zi, za, o_ref):
        o_ref[...] = jax.nn.sigmoid(
            jnp.dot(zi[...], za[...], preferred_element_type=jnp.float32))

    return pl.pallas_call(
        body,
        grid=(_N // 400,),
        in_specs=[
            pl.BlockSpec((400, 64), lambda i: (i, 0)),
            pl.BlockSpec((64, _N), lambda i: (0, 0)),
        ],
        out_specs=pl.BlockSpec((400, _N), lambda i: (i, 0)),
        out_shape=jax.ShapeDtypeStruct((_N, _N), jnp.float32),
    )(z_pad, zT)


def kernel(x, edge_index, batch, Wb, bb, gb, beb, Wm, bm, gm, bem,
           Ws, bs, gs, bes, D1, db1, D2, db2):
    n, f_in = x.shape
    e = edge_index.shape[1]
    ch = -(-e // (_C * _NW))
    e_pad = ch * _C * _NW

    x_pad = jnp.pad(x, ((0, _NPAD - n), (0, 0)))
    pad = jnp.full((e_pad - e,), n, jnp.int32)
    srcp = jnp.concatenate([edge_index[0].astype(jnp.int32), pad])
    dstp = jnp.concatenate([edge_index[1].astype(jnp.int32), pad])
    ep = jnp.stack([srcp.reshape(-1, _C), dstp.reshape(-1, _C)], axis=1)

    inv_c = 1.0 / jnp.sqrt(1.0 + 1e-4)
    gbe = (gb * inv_c).reshape(1, 64)
    gme = (gm * inv_c).reshape(1, 64)
    gse = (gs * inv_c).reshape(1, 64)
    bb2, beb2 = bb.reshape(1, 64), beb.reshape(1, 64)
    bm2, bem2 = bm.reshape(1, 64), bem.reshape(1, 64)
    bs2, bes2 = bs.reshape(1, 64), bes.reshape(1, 64)
    db1_2, db2_2 = db1.reshape(1, 64), db2.reshape(1, 128)

    noise = jax.random.normal(jax.random.key(42), (n, 64), jnp.float32)
    noise = jnp.pad(noise, ((0, _NPAD - n), (0, 0)))

    ones16 = jnp.ones((_C, 16), jnp.float32)
    z16 = jnp.zeros((_ZR, 16), jnp.float32)
    z32 = jnp.zeros((_ZR, 32), jnp.float32)
    z64 = jnp.zeros((_ZR, 64), jnp.float32)

    degp = _sc_edge_accum(16, (ch, ch), gather=False, ring=8)(ep, ones16, z16)

    h0 = _tc_h0(x_pad, Wb)
    u0a, u0b = _tc_scale(h0, degp)

    agg0 = _sc_agg_staged(32, 2 * ch, ring=16)(ep, u0a, u0b, z32)

    u1a, u1b = _tc_mid(agg0, u0a, u0b, degp, bb2, gbe, beb2, Wm, Ws)

    agg1 = _sc_agg_staged(64, 2 * ch, ring=4)(ep, u1a, u1b, z64)

    z_pad, rec_pad = _tc_head(agg1, u1a, u1b, degp, noise, bm2, gme, bem2,
                              bs2, gse, bes2, D1, db1_2, D2, db2_2)

    rec = rec_pad[:n]
    zT = z_pad[:n].T
    a_pred = _tc_decode(z_pad, zT)
    return rec, a_pred
